# baseline (speedup 1.0000x reference)
import functools

import numpy as np

import jax
import jax.numpy as jnp
from jax import lax
from jax.experimental import pallas as pl
from jax.experimental.pallas import tpu as pltpu
from jax.experimental.pallas import tpu_sc as plsc

_B = 4
_N = 128
_C = 256
_NHEAD = 8
_DH = _C // _NHEAD
_STRIDE = 16
_HP = 64
_WP = 64
_P = _HP * _WP
_K = 16
_NUM_LEVEL = 4
_NUM_POINTS = 4
_LEVEL_SHAPES = [(64, 64), (32, 32), (16, 16), (8, 8)]
_START = [0, 4096, 5120, 5376]
_LVL = 5440
_QROWS = _B * _N * _K
_OROWS = _QROWS * _NHEAD
_TERMS = 64
_TROWS = _B * _LVL * _NHEAD


def _mm_body(x_ref, w_ref, b_ref, o_ref):
    o_ref[...] = jnp.dot(x_ref[...], w_ref[...],
                         preferred_element_type=jnp.float32) + b_ref[...]


def _mm(x, w, b, bm):
    R, Kd = x.shape
    F = w.shape[1]
    return pl.pallas_call(
        _mm_body,
        grid=(R // bm,),
        in_specs=[pl.BlockSpec((bm, Kd), lambda i: (i, 0)),
                  pl.BlockSpec((Kd, F), lambda i: (0, 0)),
                  pl.BlockSpec((1, F), lambda i: (0, 0))],
        out_specs=pl.BlockSpec((bm, F), lambda i: (i, 0)),
        out_shape=jax.ShapeDtypeStruct((R, F), jnp.float32),
    )(x, w, b.reshape(1, F))


def _topk_body(c_ref, idx_ref):
    x = c_ref[0]
    iota = lax.broadcasted_iota(jnp.int32, x.shape, 1)
    cols = []
    for _ in range(_K):
        m = jnp.max(x, axis=1, keepdims=True)
        am = jnp.min(jnp.where(x == m, iota, x.shape[1]), axis=1, keepdims=True)
        cols.append(am)
        x = jnp.where(iota == am, -jnp.inf, x)
    idx_ref[0] = jnp.concatenate(cols, axis=1)


def _topk(c_t):
    return pl.pallas_call(
        _topk_body,
        grid=(_B,),
        in_specs=[pl.BlockSpec((1, _N, _P), lambda i: (i, 0, 0))],
        out_specs=pl.BlockSpec((1, _N, _K), lambda i: (i, 0, 0)),
        out_shape=jax.ShapeDtypeStruct((_B, _N, _K), jnp.int32),
    )(c_t)


def _qside_body(ql_ref, refx_ref, refy_ref, bb_ref, woff_ref, boff_ref,
                wat_ref, bat_ref, g_ref, wlf_ref, hlf_ref, wlm1_ref, hlm1_ref,
                start_ref, hvec_ref,
                i00_ref, i01_ref, i10_ref, i11_ref,
                w00_ref, w01_ref, w10_ref, w11_ref):
    q = ql_ref[...]
    offxy = jnp.dot(q, woff_ref[...], preferred_element_type=jnp.float32) + boff_ref[...]
    ox = offxy[:, :128]
    oy = offxy[:, 128:]
    al = jnp.dot(q, wat_ref[...], preferred_element_type=jnp.float32) + bat_ref[...]
    e = jnp.exp(al - jnp.max(al, axis=1, keepdims=True))
    s = jnp.dot(e, g_ref[...], preferred_element_type=jnp.float32)
    at = e / s

    x = refx_ref[...] * wlf_ref[...] + ox - 0.5
    y = refy_ref[...] * hlf_ref[...] + oy - 0.5
    x0f = jnp.floor(x)
    y0f = jnp.floor(y)
    wx = x - x0f
    wy = y - y0f
    wlm1 = wlm1_ref[...]
    hlm1 = hlm1_ref[...]
    x0 = jnp.clip(x0f.astype(jnp.int32), 0, wlm1)
    x1 = jnp.minimum(x0 + 1, wlm1)
    y0 = jnp.clip(y0f.astype(jnp.int32), 0, hlm1)
    y1 = jnp.minimum(y0 + 1, hlm1)

    wli = wlm1 + 1
    base = bb_ref[...] + start_ref[...] * 8 + hvec_ref[...]
    r0 = y0 * (wli * 8)
    r1 = y1 * (wli * 8)
    c0 = x0 * 8
    c1 = x1 * 8
    i00_ref[...] = base + r0 + c0
    i01_ref[...] = base + r0 + c1
    i10_ref[...] = base + r1 + c0
    i11_ref[...] = base + r1 + c1
    mx = 1.0 - wx
    my = 1.0 - wy
    w00_ref[...] = at * mx * my
    w01_ref[...] = at * wx * my
    w10_ref[...] = at * mx * wy
    w11_ref[...] = at * wx * wy


def _qside(ql, refx, refy, bbase, woffp, boffp, wat, bat, consts):
    bm = 1024
    g, wlf, hlf, wlm1, hlm1, start, hvec = consts
    vec_spec = pl.BlockSpec((1, 128), lambda i: (0, 0))
    outs = pl.pallas_call(
        _qside_body,
        grid=(_QROWS // bm,),
        in_specs=[pl.BlockSpec((bm, _C), lambda i: (i, 0)),
                  pl.BlockSpec((bm, 1), lambda i: (i, 0)),
                  pl.BlockSpec((bm, 1), lambda i: (i, 0)),
                  pl.BlockSpec((bm, 1), lambda i: (i, 0)),
                  pl.BlockSpec((_C, _C), lambda i: (0, 0)),
                  pl.BlockSpec((1, _C), lambda i: (0, 0)),
                  pl.BlockSpec((_C, 128), lambda i: (0, 0)),
                  vec_spec,
                  pl.BlockSpec((128, 128), lambda i: (0, 0)),
                  vec_spec, vec_spec, vec_spec, vec_spec, vec_spec, vec_spec],
        out_specs=[pl.BlockSpec((bm, 128), lambda i: (i, 0))] * 8,
        out_shape=[jax.ShapeDtypeStruct((_QROWS, 128), jnp.int32)] * 4
                + [jax.ShapeDtypeStruct((_QROWS, 128), jnp.float32)] * 4,
    )(ql, refx, refy, bbase, woffp, boffp, wat, bat, g, wlf, hlf,
      wlm1, hlm1, start, hvec)
    return outs


def _ln_in(x, g, b):
    m = jnp.mean(x, axis=1, keepdims=True)
    v = jnp.mean((x - m) ** 2, axis=1, keepdims=True)
    return (x - m) * jax.lax.rsqrt(v + 1e-5) * g + b


def _post_body(samp_ref, ql_ref, wo_ref, bo_ref, w1_ref, b1_ref, w2_ref,
               b2_ref, ln1g_ref, ln1b_ref, ln2g_ref, ln2b_ref, o_ref):
    o = jnp.dot(samp_ref[...], wo_ref[...],
                preferred_element_type=jnp.float32) + bo_ref[...]
    x = _ln_in(ql_ref[...] + o, ln1g_ref[...], ln1b_ref[...])
    h = jnp.maximum(jnp.dot(x, w1_ref[...],
                            preferred_element_type=jnp.float32) + b1_ref[...], 0.0)
    f = jnp.dot(h, w2_ref[...], preferred_element_type=jnp.float32) + b2_ref[...]
    o_ref[...] = _ln_in(x + f, ln2g_ref[...], ln2b_ref[...])


def _post(samp, ql, p, rowperm=None):
    bm = 1024
    wo = p['Wo'] if rowperm is None else p['Wo'][rowperm]
    row = lambda a: a.reshape(1, -1)
    vspec = pl.BlockSpec((1, _C), lambda i: (0, 0))
    return pl.pallas_call(
        _post_body,
        grid=(_QROWS // bm,),
        in_specs=[pl.BlockSpec((bm, _C), lambda i: (i, 0)),
                  pl.BlockSpec((bm, _C), lambda i: (i, 0)),
                  pl.BlockSpec((_C, _C), lambda i: (0, 0)),
                  vspec,
                  pl.BlockSpec((_C, 4 * _C), lambda i: (0, 0)),
                  pl.BlockSpec((1, 4 * _C), lambda i: (0, 0)),
                  pl.BlockSpec((4 * _C, _C), lambda i: (0, 0)),
                  vspec, vspec, vspec, vspec, vspec],
        out_specs=pl.BlockSpec((bm, _C), lambda i: (i, 0)),
        out_shape=jax.ShapeDtypeStruct((_QROWS, _C), jnp.float32),
    )(samp, ql, wo, row(p['bo']), p['W1'], row(p['b1']), p['W2'],
      row(p['b2']), row(p['ln1g']), row(p['ln1b']), row(p['ln2g']),
      row(p['ln2b']))


def _proj_body(qe_ref, top_ref, w1_ref, w2_ref, b_ref, uw_ref, ub_ref,
               proj_ref, unc_ref):
    proj = (jnp.dot(qe_ref[...], w1_ref[...], preferred_element_type=jnp.float32)
            + jnp.dot(top_ref[...], w2_ref[...], preferred_element_type=jnp.float32)
            + b_ref[...])
    proj_ref[...] = proj
    unc_ref[...] = jnp.sum(proj * uw_ref[...], axis=1, keepdims=True) + ub_ref[...]


def _proj(qe, top, pw, pb, uw, ub):
    bm = 1024
    return pl.pallas_call(
        _proj_body,
        grid=(_QROWS // bm,),
        in_specs=[pl.BlockSpec((bm, _C), lambda i: (i, 0)),
                  pl.BlockSpec((bm, _C), lambda i: (i, 0)),
                  pl.BlockSpec((_C, _C), lambda i: (0, 0)),
                  pl.BlockSpec((_C, _C), lambda i: (0, 0)),
                  pl.BlockSpec((1, _C), lambda i: (0, 0)),
                  pl.BlockSpec((1, _C), lambda i: (0, 0)),
                  pl.BlockSpec((1, 1), lambda i: (0, 0))],
        out_specs=[pl.BlockSpec((bm, _C), lambda i: (i, 0)),
                   pl.BlockSpec((bm, 1), lambda i: (i, 0))],
        out_shape=[jax.ShapeDtypeStruct((_QROWS, _C), jnp.float32),
                   jax.ShapeDtypeStruct((_QROWS, 1), jnp.float32)],
    )(qe, top, pw[:_C], pw[_C:], pb.reshape(1, _C), uw.reshape(1, _C),
      ub.reshape(1, 1))


def _dec_body(qq_ref, kv_ref, ghk_ref, gs_ref, bk_ref, *wrefs):
    qq = qq_ref[...]
    for layer in range(2):
        (wq, bq, wo, bo, w1, b1, w2, b2, l1g, l1b, l2g, l2b) = \
            wrefs[layer * 12:(layer + 1) * 12]
        qh = jnp.dot(qq, wq[...], preferred_element_type=jnp.float32) + bq[...]
        S = jnp.zeros((qq.shape[0], 128), jnp.float32)
        for k in range(_K):
            prod = qh * kv_ref[2 * layer, :, k, :]
            S = S + jnp.dot(prod, ghk_ref[k], preferred_element_type=jnp.float32)
        S = S * (1.0 / np.sqrt(_DH).astype(np.float32))
        e = jnp.exp(S - jnp.max(S, axis=1, keepdims=True))
        den = jnp.dot(e, gs_ref[...], preferred_element_type=jnp.float32)
        at = e / den
        o = jnp.zeros((qq.shape[0], _C), jnp.float32)
        for k in range(_K):
            ab = jnp.dot(at, bk_ref[k], preferred_element_type=jnp.float32)
            o = o + ab * kv_ref[2 * layer + 1, :, k, :]
        o = jnp.dot(o, wo[...], preferred_element_type=jnp.float32) + bo[...]
        x = _ln_in(qq + o, l1g[...], l1b[...])
        h = jnp.maximum(jnp.dot(x, w1[...],
                                preferred_element_type=jnp.float32) + b1[...], 0.0)
        f = jnp.dot(h, w2[...], preferred_element_type=jnp.float32) + b2[...]
        qq = _ln_in(x + f, l2g[...], l2b[...])
    qq_ref_out = wrefs[24]
    qq_ref_out[...] = qq


def _decoder(qq0, kv4, ghk, gs, bks, dparams):
    bm = 128
    rows = _B * _N
    row = lambda a: a.reshape(1, -1)
    vspec = pl.BlockSpec((1, _C), lambda i: (0, 0))
    in_specs = [pl.BlockSpec((bm, _C), lambda i: (i, 0)),
                pl.BlockSpec((4, bm, _K, _C), lambda i: (0, i, 0, 0)),
                pl.BlockSpec((_K, _C, 128), lambda i: (0, 0, 0)),
                pl.BlockSpec((128, 128), lambda i: (0, 0)),
                pl.BlockSpec((_K, 128, _C), lambda i: (0, 0, 0))]
    args = [qq0, kv4, ghk, gs, bks]
    for p in dparams:
        in_specs += [pl.BlockSpec((_C, _C), lambda i: (0, 0)), vspec,
                     pl.BlockSpec((_C, _C), lambda i: (0, 0)), vspec,
                     pl.BlockSpec((_C, 4 * _C), lambda i: (0, 0)),
                     pl.BlockSpec((1, 4 * _C), lambda i: (0, 0)),
                     pl.BlockSpec((4 * _C, _C), lambda i: (0, 0)),
                     vspec, vspec, vspec, vspec, vspec]
        args += [p['Wq'], row(p['bq']), p['Wo'], row(p['bo']), p['W1'],
                 row(p['b1']), p['W2'], row(p['b2']), row(p['ln1g']),
                 row(p['ln1b']), row(p['ln2g']), row(p['ln2b'])]
    return pl.pallas_call(
        _dec_body,
        grid=(rows // bm,),
        in_specs=in_specs,
        out_specs=pl.BlockSpec((bm, _C), lambda i: (i, 0)),
        out_shape=jax.ShapeDtypeStruct((rows, _C), jnp.float32),
    )(*args)


_NW = 32
_CB = 8
_GPW = 1024
_RPW = 2048


_NCH = _GPW // _CB


def _bag_body(table, idxh, wh, out, idx_v, w_v, grows, out_v, sem_g, sem_s):
    wid = lax.axis_index("s") * 2 + lax.axis_index("c")

    def stage_idx(cc):
        @pl.when(cc < _NCH)
        def _():
            s4 = lax.rem(cc, 4)
            pltpu.async_copy(idxh.at[wid, pl.ds(cc * _CB, _CB), :],
                             idx_v.at[s4], sem_s)
            pltpu.async_copy(wh.at[wid, pl.ds(cc * _CB, _CB), :],
                             w_v.at[s4], sem_s)

    def wait_stage(cc):
        @pl.when(cc < _NCH)
        def _():
            s4 = lax.rem(cc, 4)
            pltpu.make_async_copy(idxh.at[wid, pl.ds(cc * _CB, _CB), :],
                                  idx_v.at[s4], sem_s).wait()
            pltpu.make_async_copy(wh.at[wid, pl.ds(cc * _CB, _CB), :],
                                  w_v.at[s4], sem_s).wait()

    def fire_gathers(cc):
        @pl.when(cc < _NCH)
        def _():
            s4 = lax.rem(cc, 4)
            s2 = lax.rem(cc, 2)
            for g in range(_CB):
                pltpu.async_copy(table.at[idx_v.at[s4, g]],
                                 grows.at[s2, g], sem_g)

    stage_idx(0)
    stage_idx(1)
    wait_stage(0)
    fire_gathers(0)

    def outer(c, carry):
        s4 = lax.rem(c, 4)
        s2 = lax.rem(c, 2)
        stage_idx(c + 2)
        wait_stage(c + 1)
        fire_gathers(c + 1)
        for g in range(_CB):
            pltpu.make_async_copy(table.at[idx_v.at[s4, g]],
                                  grows.at[s2, g], sem_g).wait()
        for g in range(_CB):
            for half in range(2):
                off = half * 64
                wvs = [w_v[s4, g, pl.ds(off + k * 16, 16)] for k in range(4)]
                z = jnp.zeros((16,), jnp.float32)
                a0 = [z, z, z, z]
                a1 = [z, z, z, z]
                for j in range(64):
                    k = j // 16
                    s = wvs[k][j % 16]
                    a0[k] = a0[k] + s * grows[s2, g, off + j, pl.ds(0, 16)]
                    a1[k] = a1[k] + s * grows[s2, g, off + j, pl.ds(16, 16)]
                out_v[2 * g + half, pl.ds(0, 16)] = (a0[0] + a0[1]) + (a0[2] + a0[3])
                out_v[2 * g + half, pl.ds(16, 16)] = (a1[0] + a1[1]) + (a1[2] + a1[3])
        pltpu.sync_copy(out_v,
                        out.at[pl.ds(wid * _RPW + c * 2 * _CB, 2 * _CB), :])
        return carry

    lax.fori_loop(0, _NCH, outer, None)


@functools.partial(
    pl.kernel,
    mesh=plsc.VectorSubcoreMesh(core_axis_name="c", subcore_axis_name="s"),
    compiler_params=pltpu.CompilerParams(use_tc_tiling_on_sc=False),
    out_type=jax.ShapeDtypeStruct((_OROWS, _DH), jnp.float32),
    scratch_types=[
        pltpu.VMEM((4, _CB, 128), jnp.int32),
        pltpu.VMEM((4, _CB, 128), jnp.float32),
        pltpu.VMEM((2, _CB, 128, _DH), jnp.float32),
        pltpu.VMEM((2 * _CB, _DH), jnp.float32),
        pltpu.SemaphoreType.DMA,
        pltpu.SemaphoreType.DMA,
    ],
)
def _bag(table, idxh, wh, out, idx_v, w_v, grows, out_v, sem_g, sem_s):
    _bag_body(table, idxh, wh, out, idx_v, w_v, grows, out_v, sem_g, sem_s)


_UNPACK_ROWPERM = np.concatenate(
    [h * 32 + np.concatenate([np.arange(16) * 2, np.arange(16) * 2 + 1])
     for h in range(_NHEAD)])


def _lane_consts():
    lanes = np.arange(128)
    h = lanes // 16
    l = (lanes % 16) // 4
    wl = np.array([s[1] for s in _LEVEL_SHAPES], np.float32)[l]
    hl = np.array([s[0] for s in _LEVEL_SHAPES], np.float32)[l]
    start = np.array(_START, np.int32)[l]
    g = (lanes[:, None] // 16 == lanes[None, :] // 16).astype(np.float32)
    return (jnp.asarray(g),
            jnp.asarray(wl.reshape(1, 128)),
            jnp.asarray(hl.reshape(1, 128)),
            jnp.asarray((wl - 1).astype(np.int32).reshape(1, 128)),
            jnp.asarray((hl - 1).astype(np.int32).reshape(1, 128)),
            jnp.asarray(start.reshape(1, 128)),
            jnp.asarray(h.astype(np.int32).reshape(1, 128)))


def _dec_consts():
    i = np.arange(_C)
    ghk = np.zeros((_K, _C, 128), np.float32)
    for k in range(_K):
        ghk[k, i, (i // _DH) * _K + k] = 1.0
    lanes = np.arange(128)
    gs = (lanes[:, None] // _K == lanes[None, :] // _K).astype(np.float32)
    bk = np.zeros((_K, 128, _C), np.float32)
    for k in range(_K):
        for h in range(_NHEAD):
            bk[k, h * _K + k, h * _DH:(h + 1) * _DH] = 1.0
    return jnp.asarray(ghk), jnp.asarray(gs), jnp.asarray(bk)


def kernel(q_t, h_t, c_t, params):
    idx = _topk(c_t)
    xs = (idx % _WP).astype(jnp.float32) * _STRIDE + _STRIDE / 2.0
    ys = (idx // _WP).astype(jnp.float32) * _STRIDE + _STRIDE / 2.0
    tloc = jnp.stack([xs, ys], -1)
    norm = jnp.clip(tloc / 1024.0, 0.0, 1.0)

    img = h_t.reshape(_B, _HP, _WP, _C)
    f1 = h_t
    f2 = img.reshape(_B, 32, 2, 32, 2, _C).mean((2, 4)).reshape(_B, 1024, _C)
    f3 = img.reshape(_B, 16, 4, 16, 4, _C).mean((2, 4)).reshape(_B, 256, _C)
    f4 = f1.reshape(_B, 512, 8, 32, 8).mean((2, 4)).reshape(_B, _C, 64).transpose(0, 2, 1)
    fs = jnp.concatenate([f1, f2, f3, f4], 1).reshape(_B * _LVL, _C)

    enc = params['enc']
    wv = jnp.concatenate([enc[0]['Wv'], enc[1]['Wv']], axis=1)
    bv = jnp.concatenate([enc[0]['bv'], enc[1]['bv']], axis=0)
    vall = _mm(fs, wv, bv, bm=1360)

    consts = _lane_consts()
    refx = norm[..., 0].reshape(_QROWS, 1)
    refy = norm[..., 1].reshape(_QROWS, 1)
    bbase = jnp.repeat(jnp.arange(_B, dtype=jnp.int32) * (_LVL * _NHEAD),
                       _N * _K).reshape(_QROWS, 1)
    qe = jnp.broadcast_to(q_t[:, :, None, :], (_B, _N, _K, _C)).reshape(_QROWS, _C)

    perm = np.array([h * 32 + l * 8 + p * 2 + xy
                     for xy in range(2) for h in range(_NHEAD)
                     for l in range(_NUM_LEVEL) for p in range(_NUM_POINTS)])
    ql = qe
    for li, p in enumerate(enc):
        woffp = p['Woff'][:, perm]
        boffp = p['boff'][perm].reshape(1, 256)
        outs = _qside(ql, refx, refy, bbase, woffp, boffp, p['Wat'],
                      p['bat'].reshape(1, 128), consts)
        i00, i01, i10, i11, w00, w01, w10, w11 = outs
        idxs = jnp.concatenate(
            [a.reshape(_OROWS, 16) for a in (i00, i01, i10, i11)], axis=1)
        ws = jnp.concatenate(
            [a.reshape(_OROWS, 16) for a in (w00, w01, w10, w11)], axis=1)
        table = vall[:, li * _C:(li + 1) * _C].reshape(_TROWS, _DH)
        samp = _bag(table, idxs.reshape(_NW, _GPW, 128),
                    ws.reshape(_NW, _GPW, 128))
        ql = _post(samp.reshape(_QROWS, _C), ql, p)

    proj, unc = _proj(qe, ql, params['proj_W'], params['proj_b'],
                      params['unc_W'], params['unc_b'])

    dec = params['dec']
    wkv = jnp.concatenate([dec[0]['Wk'], dec[0]['Wv'],
                           dec[1]['Wk'], dec[1]['Wv']], axis=1)
    bkv = jnp.concatenate([dec[0]['bk'], dec[0]['bv'],
                           dec[1]['bk'], dec[1]['bv']], axis=0)
    khvh = _mm(proj, wkv, bkv, bm=1024)
    kv4 = khvh.reshape(_B * _N, _K, 4, _C).transpose(2, 0, 1, 3)
    ghk, gs, bks = _dec_consts()
    qq = _decoder(q_t.reshape(_B * _N, _C), kv4, ghk, gs, bks, dec)

    fin = jnp.concatenate([q_t.reshape(_B * _N, _C), qq], axis=1)
    out = _mm(fin, params['fusion_W'], params['fusion_b'], bm=512)

    return (out.reshape(_B, _N, _C), unc.reshape(_B, _N, _K), tloc)


# speedup vs baseline: 1.0062x; 1.0062x over previous
import functools

import numpy as np

import jax
import jax.numpy as jnp
from jax import lax
from jax.experimental import pallas as pl
from jax.experimental.pallas import tpu as pltpu
from jax.experimental.pallas import tpu_sc as plsc

_B = 4
_N = 128
_C = 256
_NHEAD = 8
_DH = _C // _NHEAD
_STRIDE = 16
_HP = 64
_WP = 64
_P = _HP * _WP
_K = 16
_NUM_LEVEL = 4
_NUM_POINTS = 4
_LEVEL_SHAPES = [(64, 64), (32, 32), (16, 16), (8, 8)]
_START = [0, 4096, 5120, 5376]
_LVL = 5440
_QROWS = _B * _N * _K
_OROWS = _QROWS * _NHEAD
_TERMS = 64
_TROWS = _B * _LVL * _NHEAD


def _mm_body(x_ref, w_ref, b_ref, o_ref):
    o_ref[...] = jnp.dot(x_ref[...], w_ref[...],
                         preferred_element_type=jnp.float32) + b_ref[...]


def _mm(x, w, b, bm):
    R, Kd = x.shape
    F = w.shape[1]
    return pl.pallas_call(
        _mm_body,
        grid=(R // bm,),
        in_specs=[pl.BlockSpec((bm, Kd), lambda i: (i, 0)),
                  pl.BlockSpec((Kd, F), lambda i: (0, 0)),
                  pl.BlockSpec((1, F), lambda i: (0, 0))],
        out_specs=pl.BlockSpec((bm, F), lambda i: (i, 0)),
        out_shape=jax.ShapeDtypeStruct((R, F), jnp.float32),
    )(x, w, b.reshape(1, F))


def _topk_body(c_ref, idx_ref):
    x = c_ref[0]
    iota = lax.broadcasted_iota(jnp.int32, x.shape, 1)
    cols = []
    for _ in range(_K):
        m = jnp.max(x, axis=1, keepdims=True)
        am = jnp.min(jnp.where(x == m, iota, x.shape[1]), axis=1, keepdims=True)
        cols.append(am)
        x = jnp.where(iota == am, -jnp.inf, x)
    idx_ref[0] = jnp.concatenate(cols, axis=1)


def _topk(c_t):
    return pl.pallas_call(
        _topk_body,
        grid=(_B,),
        in_specs=[pl.BlockSpec((1, _N, _P), lambda i: (i, 0, 0))],
        out_specs=pl.BlockSpec((1, _N, _K), lambda i: (i, 0, 0)),
        out_shape=jax.ShapeDtypeStruct((_B, _N, _K), jnp.int32),
    )(c_t)


def _qside_body(ql_ref, refx_ref, refy_ref, bb_ref, woff_ref, boff_ref,
                wat_ref, bat_ref, g_ref, wlf_ref, hlf_ref, wlm1_ref, hlm1_ref,
                start_ref, hvec_ref,
                i00_ref, i01_ref, i10_ref, i11_ref,
                w00_ref, w01_ref, w10_ref, w11_ref):
    q = ql_ref[...]
    offxy = jnp.dot(q, woff_ref[...], preferred_element_type=jnp.float32) + boff_ref[...]
    ox = offxy[:, :128]
    oy = offxy[:, 128:]
    al = jnp.dot(q, wat_ref[...], preferred_element_type=jnp.float32) + bat_ref[...]
    e = jnp.exp(al - jnp.max(al, axis=1, keepdims=True))
    s = jnp.dot(e, g_ref[...], preferred_element_type=jnp.float32)
    at = e / s

    x = refx_ref[...] * wlf_ref[...] + ox - 0.5
    y = refy_ref[...] * hlf_ref[...] + oy - 0.5
    x0f = jnp.floor(x)
    y0f = jnp.floor(y)
    wx = x - x0f
    wy = y - y0f
    wlm1 = wlm1_ref[...]
    hlm1 = hlm1_ref[...]
    x0 = jnp.clip(x0f.astype(jnp.int32), 0, wlm1)
    x1 = jnp.minimum(x0 + 1, wlm1)
    y0 = jnp.clip(y0f.astype(jnp.int32), 0, hlm1)
    y1 = jnp.minimum(y0 + 1, hlm1)

    wli = wlm1 + 1
    base = bb_ref[...] + start_ref[...] * 8 + hvec_ref[...]
    r0 = y0 * (wli * 8)
    r1 = y1 * (wli * 8)
    c0 = x0 * 8
    c1 = x1 * 8
    i00_ref[...] = base + r0 + c0
    i01_ref[...] = base + r0 + c1
    i10_ref[...] = base + r1 + c0
    i11_ref[...] = base + r1 + c1
    mx = 1.0 - wx
    my = 1.0 - wy
    w00_ref[...] = at * mx * my
    w01_ref[...] = at * wx * my
    w10_ref[...] = at * mx * wy
    w11_ref[...] = at * wx * wy


def _qside(ql, refx, refy, bbase, woffp, boffp, wat, bat, consts):
    bm = 1024
    g, wlf, hlf, wlm1, hlm1, start, hvec = consts
    vec_spec = pl.BlockSpec((1, 128), lambda i: (0, 0))
    outs = pl.pallas_call(
        _qside_body,
        grid=(_QROWS // bm,),
        in_specs=[pl.BlockSpec((bm, _C), lambda i: (i, 0)),
                  pl.BlockSpec((bm, 1), lambda i: (i, 0)),
                  pl.BlockSpec((bm, 1), lambda i: (i, 0)),
                  pl.BlockSpec((bm, 1), lambda i: (i, 0)),
                  pl.BlockSpec((_C, _C), lambda i: (0, 0)),
                  pl.BlockSpec((1, _C), lambda i: (0, 0)),
                  pl.BlockSpec((_C, 128), lambda i: (0, 0)),
                  vec_spec,
                  pl.BlockSpec((128, 128), lambda i: (0, 0)),
                  vec_spec, vec_spec, vec_spec, vec_spec, vec_spec, vec_spec],
        out_specs=[pl.BlockSpec((bm, 128), lambda i: (i, 0))] * 8,
        out_shape=[jax.ShapeDtypeStruct((_QROWS, 128), jnp.int32)] * 4
                + [jax.ShapeDtypeStruct((_QROWS, 128), jnp.float32)] * 4,
    )(ql, refx, refy, bbase, woffp, boffp, wat, bat, g, wlf, hlf,
      wlm1, hlm1, start, hvec)
    return outs


def _ln_in(x, g, b):
    m = jnp.mean(x, axis=1, keepdims=True)
    v = jnp.mean((x - m) ** 2, axis=1, keepdims=True)
    return (x - m) * jax.lax.rsqrt(v + 1e-5) * g + b


def _post_body(samp_ref, ql_ref, wo_ref, bo_ref, w1_ref, b1_ref, w2_ref,
               b2_ref, ln1g_ref, ln1b_ref, ln2g_ref, ln2b_ref, o_ref):
    o = jnp.dot(samp_ref[...], wo_ref[...],
                preferred_element_type=jnp.float32) + bo_ref[...]
    x = _ln_in(ql_ref[...] + o, ln1g_ref[...], ln1b_ref[...])
    h = jnp.maximum(jnp.dot(x, w1_ref[...],
                            preferred_element_type=jnp.float32) + b1_ref[...], 0.0)
    f = jnp.dot(h, w2_ref[...], preferred_element_type=jnp.float32) + b2_ref[...]
    o_ref[...] = _ln_in(x + f, ln2g_ref[...], ln2b_ref[...])


def _post(samp, ql, p, rowperm=None):
    bm = 1024
    wo = p['Wo'] if rowperm is None else p['Wo'][rowperm]
    row = lambda a: a.reshape(1, -1)
    vspec = pl.BlockSpec((1, _C), lambda i: (0, 0))
    return pl.pallas_call(
        _post_body,
        grid=(_QROWS // bm,),
        in_specs=[pl.BlockSpec((bm, _C), lambda i: (i, 0)),
                  pl.BlockSpec((bm, _C), lambda i: (i, 0)),
                  pl.BlockSpec((_C, _C), lambda i: (0, 0)),
                  vspec,
                  pl.BlockSpec((_C, 4 * _C), lambda i: (0, 0)),
                  pl.BlockSpec((1, 4 * _C), lambda i: (0, 0)),
                  pl.BlockSpec((4 * _C, _C), lambda i: (0, 0)),
                  vspec, vspec, vspec, vspec, vspec],
        out_specs=pl.BlockSpec((bm, _C), lambda i: (i, 0)),
        out_shape=jax.ShapeDtypeStruct((_QROWS, _C), jnp.float32),
    )(samp, ql, wo, row(p['bo']), p['W1'], row(p['b1']), p['W2'],
      row(p['b2']), row(p['ln1g']), row(p['ln1b']), row(p['ln2g']),
      row(p['ln2b']))


def _proj_body(qe_ref, top_ref, w1_ref, w2_ref, b_ref, uw_ref, ub_ref,
               proj_ref, unc_ref):
    proj = (jnp.dot(qe_ref[...], w1_ref[...], preferred_element_type=jnp.float32)
            + jnp.dot(top_ref[...], w2_ref[...], preferred_element_type=jnp.float32)
            + b_ref[...])
    proj_ref[...] = proj
    unc_ref[...] = jnp.sum(proj * uw_ref[...], axis=1, keepdims=True) + ub_ref[...]


def _proj(qe, top, pw, pb, uw, ub):
    bm = 1024
    return pl.pallas_call(
        _proj_body,
        grid=(_QROWS // bm,),
        in_specs=[pl.BlockSpec((bm, _C), lambda i: (i, 0)),
                  pl.BlockSpec((bm, _C), lambda i: (i, 0)),
                  pl.BlockSpec((_C, _C), lambda i: (0, 0)),
                  pl.BlockSpec((_C, _C), lambda i: (0, 0)),
                  pl.BlockSpec((1, _C), lambda i: (0, 0)),
                  pl.BlockSpec((1, _C), lambda i: (0, 0)),
                  pl.BlockSpec((1, 1), lambda i: (0, 0))],
        out_specs=[pl.BlockSpec((bm, _C), lambda i: (i, 0)),
                   pl.BlockSpec((bm, 1), lambda i: (i, 0))],
        out_shape=[jax.ShapeDtypeStruct((_QROWS, _C), jnp.float32),
                   jax.ShapeDtypeStruct((_QROWS, 1), jnp.float32)],
    )(qe, top, pw[:_C], pw[_C:], pb.reshape(1, _C), uw.reshape(1, _C),
      ub.reshape(1, 1))


def _dec_body(qq_ref, kv_ref, ghk_ref, gs_ref, bk_ref, *wrefs):
    qq = qq_ref[...]
    for layer in range(2):
        (wq, bq, wo, bo, w1, b1, w2, b2, l1g, l1b, l2g, l2b) = \
            wrefs[layer * 12:(layer + 1) * 12]
        qh = jnp.dot(qq, wq[...], preferred_element_type=jnp.float32) + bq[...]
        S = jnp.zeros((qq.shape[0], 128), jnp.float32)
        for k in range(_K):
            prod = qh * kv_ref[2 * layer, :, k, :]
            S = S + jnp.dot(prod, ghk_ref[k], preferred_element_type=jnp.float32)
        S = S * (1.0 / np.sqrt(_DH).astype(np.float32))
        e = jnp.exp(S - jnp.max(S, axis=1, keepdims=True))
        den = jnp.dot(e, gs_ref[...], preferred_element_type=jnp.float32)
        at = e / den
        o = jnp.zeros((qq.shape[0], _C), jnp.float32)
        for k in range(_K):
            ab = jnp.dot(at, bk_ref[k], preferred_element_type=jnp.float32)
            o = o + ab * kv_ref[2 * layer + 1, :, k, :]
        o = jnp.dot(o, wo[...], preferred_element_type=jnp.float32) + bo[...]
        x = _ln_in(qq + o, l1g[...], l1b[...])
        h = jnp.maximum(jnp.dot(x, w1[...],
                                preferred_element_type=jnp.float32) + b1[...], 0.0)
        f = jnp.dot(h, w2[...], preferred_element_type=jnp.float32) + b2[...]
        qq = _ln_in(x + f, l2g[...], l2b[...])
    qq_ref_out = wrefs[24]
    qq_ref_out[...] = qq


def _decoder(qq0, kv4, ghk, gs, bks, dparams):
    bm = 128
    rows = _B * _N
    row = lambda a: a.reshape(1, -1)
    vspec = pl.BlockSpec((1, _C), lambda i: (0, 0))
    in_specs = [pl.BlockSpec((bm, _C), lambda i: (i, 0)),
                pl.BlockSpec((4, bm, _K, _C), lambda i: (0, i, 0, 0)),
                pl.BlockSpec((_K, _C, 128), lambda i: (0, 0, 0)),
                pl.BlockSpec((128, 128), lambda i: (0, 0)),
                pl.BlockSpec((_K, 128, _C), lambda i: (0, 0, 0))]
    args = [qq0, kv4, ghk, gs, bks]
    for p in dparams:
        in_specs += [pl.BlockSpec((_C, _C), lambda i: (0, 0)), vspec,
                     pl.BlockSpec((_C, _C), lambda i: (0, 0)), vspec,
                     pl.BlockSpec((_C, 4 * _C), lambda i: (0, 0)),
                     pl.BlockSpec((1, 4 * _C), lambda i: (0, 0)),
                     pl.BlockSpec((4 * _C, _C), lambda i: (0, 0)),
                     vspec, vspec, vspec, vspec, vspec]
        args += [p['Wq'], row(p['bq']), p['Wo'], row(p['bo']), p['W1'],
                 row(p['b1']), p['W2'], row(p['b2']), row(p['ln1g']),
                 row(p['ln1b']), row(p['ln2g']), row(p['ln2b'])]
    return pl.pallas_call(
        _dec_body,
        grid=(rows // bm,),
        in_specs=in_specs,
        out_specs=pl.BlockSpec((bm, _C), lambda i: (i, 0)),
        out_shape=jax.ShapeDtypeStruct((rows, _C), jnp.float32),
    )(*args)


_NW = 32
_CB = 8
_GPW = 1024
_RPW = 2048


_NCH = _GPW // _CB


def _bag_body(table, idxh, wh, out, idx_v, w_v, grows, out_v, sem_g, sem_s,
              sem_o):
    wid = lax.axis_index("s") * 2 + lax.axis_index("c")

    def stage_idx(cc):
        @pl.when(cc < _NCH)
        def _():
            s4 = lax.rem(cc, 4)
            pltpu.async_copy(idxh.at[wid, pl.ds(cc * _CB, _CB), :],
                             idx_v.at[s4], sem_s)
            pltpu.async_copy(wh.at[wid, pl.ds(cc * _CB, _CB), :],
                             w_v.at[s4], sem_s)

    def wait_stage(cc):
        @pl.when(cc < _NCH)
        def _():
            s4 = lax.rem(cc, 4)
            pltpu.make_async_copy(idxh.at[wid, pl.ds(cc * _CB, _CB), :],
                                  idx_v.at[s4], sem_s).wait()
            pltpu.make_async_copy(wh.at[wid, pl.ds(cc * _CB, _CB), :],
                                  w_v.at[s4], sem_s).wait()

    def fire_gathers(cc):
        @pl.when(cc < _NCH)
        def _():
            s4 = lax.rem(cc, 4)
            s2 = lax.rem(cc, 2)
            for g in range(_CB):
                pltpu.async_copy(table.at[idx_v.at[s4, g]],
                                 grows.at[s2, g], sem_g)

    stage_idx(0)
    stage_idx(1)
    wait_stage(0)
    fire_gathers(0)

    def outer(c, carry):
        s4 = lax.rem(c, 4)
        s2 = lax.rem(c, 2)
        stage_idx(c + 2)
        wait_stage(c + 1)
        fire_gathers(c + 1)
        for g in range(_CB):
            pltpu.make_async_copy(table.at[idx_v.at[s4, g]],
                                  grows.at[s2, g], sem_g).wait()

        @pl.when(c >= 2)
        def _():
            pltpu.make_async_copy(
                out_v.at[s2],
                out.at[pl.ds(wid * _RPW + (c - 2) * 2 * _CB, 2 * _CB), :],
                sem_o).wait()
        for g in range(_CB):
            for half in range(2):
                off = half * 64
                wvs = [w_v[s4, g, pl.ds(off + k * 16, 16)] for k in range(4)]
                z = jnp.zeros((16,), jnp.float32)
                a0 = [z, z, z, z]
                a1 = [z, z, z, z]
                for j in range(64):
                    k = j // 16
                    s = wvs[k][j % 16]
                    a0[k] = a0[k] + s * grows[s2, g, off + j, pl.ds(0, 16)]
                    a1[k] = a1[k] + s * grows[s2, g, off + j, pl.ds(16, 16)]
                out_v[s2, 2 * g + half, pl.ds(0, 16)] = (a0[0] + a0[1]) + (a0[2] + a0[3])
                out_v[s2, 2 * g + half, pl.ds(16, 16)] = (a1[0] + a1[1]) + (a1[2] + a1[3])
        pltpu.async_copy(out_v.at[s2],
                         out.at[pl.ds(wid * _RPW + c * 2 * _CB, 2 * _CB), :],
                         sem_o)
        return carry

    lax.fori_loop(0, _NCH, outer, None)
    for cc in (_NCH - 2, _NCH - 1):
        pltpu.make_async_copy(
            out_v.at[cc % 2],
            out.at[pl.ds(wid * _RPW + cc * 2 * _CB, 2 * _CB), :],
            sem_o).wait()


@functools.partial(
    pl.kernel,
    mesh=plsc.VectorSubcoreMesh(core_axis_name="c", subcore_axis_name="s"),
    compiler_params=pltpu.CompilerParams(use_tc_tiling_on_sc=False),
    out_type=jax.ShapeDtypeStruct((_OROWS, _DH), jnp.float32),
    scratch_types=[
        pltpu.VMEM((4, _CB, 128), jnp.int32),
        pltpu.VMEM((4, _CB, 128), jnp.float32),
        pltpu.VMEM((2, _CB, 128, _DH), jnp.float32),
        pltpu.VMEM((2, 2 * _CB, _DH), jnp.float32),
        pltpu.SemaphoreType.DMA,
        pltpu.SemaphoreType.DMA,
        pltpu.SemaphoreType.DMA,
    ],
)
def _bag(table, idxh, wh, out, idx_v, w_v, grows, out_v, sem_g, sem_s, sem_o):
    _bag_body(table, idxh, wh, out, idx_v, w_v, grows, out_v, sem_g, sem_s,
              sem_o)


_UNPACK_ROWPERM = np.concatenate(
    [h * 32 + np.concatenate([np.arange(16) * 2, np.arange(16) * 2 + 1])
     for h in range(_NHEAD)])


def _lane_consts():
    lanes = np.arange(128)
    h = lanes // 16
    l = (lanes % 16) // 4
    wl = np.array([s[1] for s in _LEVEL_SHAPES], np.float32)[l]
    hl = np.array([s[0] for s in _LEVEL_SHAPES], np.float32)[l]
    start = np.array(_START, np.int32)[l]
    g = (lanes[:, None] // 16 == lanes[None, :] // 16).astype(np.float32)
    return (jnp.asarray(g),
            jnp.asarray(wl.reshape(1, 128)),
            jnp.asarray(hl.reshape(1, 128)),
            jnp.asarray((wl - 1).astype(np.int32).reshape(1, 128)),
            jnp.asarray((hl - 1).astype(np.int32).reshape(1, 128)),
            jnp.asarray(start.reshape(1, 128)),
            jnp.asarray(h.astype(np.int32).reshape(1, 128)))


def _dec_consts():
    i = np.arange(_C)
    ghk = np.zeros((_K, _C, 128), np.float32)
    for k in range(_K):
        ghk[k, i, (i // _DH) * _K + k] = 1.0
    lanes = np.arange(128)
    gs = (lanes[:, None] // _K == lanes[None, :] // _K).astype(np.float32)
    bk = np.zeros((_K, 128, _C), np.float32)
    for k in range(_K):
        for h in range(_NHEAD):
            bk[k, h * _K + k, h * _DH:(h + 1) * _DH] = 1.0
    return jnp.asarray(ghk), jnp.asarray(gs), jnp.asarray(bk)


def kernel(q_t, h_t, c_t, params):
    idx = _topk(c_t)
    xs = (idx % _WP).astype(jnp.float32) * _STRIDE + _STRIDE / 2.0
    ys = (idx // _WP).astype(jnp.float32) * _STRIDE + _STRIDE / 2.0
    tloc = jnp.stack([xs, ys], -1)
    norm = jnp.clip(tloc / 1024.0, 0.0, 1.0)

    img = h_t.reshape(_B, _HP, _WP, _C)
    f1 = h_t
    f2 = img.reshape(_B, 32, 2, 32, 2, _C).mean((2, 4)).reshape(_B, 1024, _C)
    f3 = img.reshape(_B, 16, 4, 16, 4, _C).mean((2, 4)).reshape(_B, 256, _C)
    f4 = f1.reshape(_B, 512, 8, 32, 8).mean((2, 4)).reshape(_B, _C, 64).transpose(0, 2, 1)
    fs = jnp.concatenate([f1, f2, f3, f4], 1).reshape(_B * _LVL, _C)

    enc = params['enc']
    wv = jnp.concatenate([enc[0]['Wv'], enc[1]['Wv']], axis=1)
    bv = jnp.concatenate([enc[0]['bv'], enc[1]['bv']], axis=0)
    vall = _mm(fs, wv, bv, bm=1360)

    consts = _lane_consts()
    refx = norm[..., 0].reshape(_QROWS, 1)
    refy = norm[..., 1].reshape(_QROWS, 1)
    bbase = jnp.repeat(jnp.arange(_B, dtype=jnp.int32) * (_LVL * _NHEAD),
                       _N * _K).reshape(_QROWS, 1)
    qe = jnp.broadcast_to(q_t[:, :, None, :], (_B, _N, _K, _C)).reshape(_QROWS, _C)

    perm = np.array([h * 32 + l * 8 + p * 2 + xy
                     for xy in range(2) for h in range(_NHEAD)
                     for l in range(_NUM_LEVEL) for p in range(_NUM_POINTS)])
    ql = qe
    for li, p in enumerate(enc):
        woffp = p['Woff'][:, perm]
        boffp = p['boff'][perm].reshape(1, 256)
        outs = _qside(ql, refx, refy, bbase, woffp, boffp, p['Wat'],
                      p['bat'].reshape(1, 128), consts)
        i00, i01, i10, i11, w00, w01, w10, w11 = outs
        idxs = jnp.concatenate(
            [a.reshape(_OROWS, 16) for a in (i00, i01, i10, i11)], axis=1)
        ws = jnp.concatenate(
            [a.reshape(_OROWS, 16) for a in (w00, w01, w10, w11)], axis=1)
        table = vall[:, li * _C:(li + 1) * _C].reshape(_TROWS, _DH)
        samp = _bag(table, idxs.reshape(_NW, _GPW, 128),
                    ws.reshape(_NW, _GPW, 128))
        ql = _post(samp.reshape(_QROWS, _C), ql, p)

    proj, unc = _proj(qe, ql, params['proj_W'], params['proj_b'],
                      params['unc_W'], params['unc_b'])

    dec = params['dec']
    wkv = jnp.concatenate([dec[0]['Wk'], dec[0]['Wv'],
                           dec[1]['Wk'], dec[1]['Wv']], axis=1)
    bkv = jnp.concatenate([dec[0]['bk'], dec[0]['bv'],
                           dec[1]['bk'], dec[1]['bv']], axis=0)
    khvh = _mm(proj, wkv, bkv, bm=1024)
    kv4 = khvh.reshape(_B * _N, _K, 4, _C).transpose(2, 0, 1, 3)
    ghk, gs, bks = _dec_consts()
    qq = _decoder(q_t.reshape(_B * _N, _C), kv4, ghk, gs, bks, dec)

    fin = jnp.concatenate([q_t.reshape(_B * _N, _C), qq], axis=1)
    out = _mm(fin, params['fusion_W'], params['fusion_b'], bm=512)

    return (out.reshape(_B, _N, _C), unc.reshape(_B, _N, _K), tloc)


# speedup vs baseline: 1.0118x; 1.0056x over previous
import functools

import numpy as np

import jax
import jax.numpy as jnp
from jax import lax
from jax.experimental import pallas as pl
from jax.experimental.pallas import tpu as pltpu
from jax.experimental.pallas import tpu_sc as plsc

_B = 4
_N = 128
_C = 256
_NHEAD = 8
_DH = _C // _NHEAD
_STRIDE = 16
_HP = 64
_WP = 64
_P = _HP * _WP
_K = 16
_NUM_LEVEL = 4
_NUM_POINTS = 4
_LEVEL_SHAPES = [(64, 64), (32, 32), (16, 16), (8, 8)]
_START = [0, 4096, 5120, 5376]
_LVL = 5440
_QROWS = _B * _N * _K
_OROWS = _QROWS * _NHEAD
_TERMS = 64
_TROWS = _B * _LVL * _NHEAD


def _mm_body(x_ref, w_ref, b_ref, o_ref):
    o_ref[...] = jnp.dot(x_ref[...], w_ref[...],
                         preferred_element_type=jnp.float32) + b_ref[...]


def _mm(x, w, b, bm):
    R, Kd = x.shape
    F = w.shape[1]
    return pl.pallas_call(
        _mm_body,
        grid=(R // bm,),
        in_specs=[pl.BlockSpec((bm, Kd), lambda i: (i, 0)),
                  pl.BlockSpec((Kd, F), lambda i: (0, 0)),
                  pl.BlockSpec((1, F), lambda i: (0, 0))],
        out_specs=pl.BlockSpec((bm, F), lambda i: (i, 0)),
        out_shape=jax.ShapeDtypeStruct((R, F), jnp.float32),
    )(x, w, b.reshape(1, F))


def _topk_body(c_ref, idx_ref):
    x = c_ref[0]
    iota = lax.broadcasted_iota(jnp.int32, x.shape, 1)
    cols = []
    for _ in range(_K):
        m = jnp.max(x, axis=1, keepdims=True)
        am = jnp.min(jnp.where(x == m, iota, x.shape[1]), axis=1, keepdims=True)
        cols.append(am)
        x = jnp.where(iota == am, -jnp.inf, x)
    idx_ref[0] = jnp.concatenate(cols, axis=1)


def _topk(c_t):
    return pl.pallas_call(
        _topk_body,
        grid=(_B,),
        in_specs=[pl.BlockSpec((1, _N, _P), lambda i: (i, 0, 0))],
        out_specs=pl.BlockSpec((1, _N, _K), lambda i: (i, 0, 0)),
        out_shape=jax.ShapeDtypeStruct((_B, _N, _K), jnp.int32),
    )(c_t)


def _qside_body(ql_ref, refx_ref, refy_ref, bb_ref, woff_ref, boff_ref,
                wat_ref, bat_ref, g_ref, wlf_ref, hlf_ref, wlm1_ref, hlm1_ref,
                start_ref, hvec_ref,
                i00_ref, i01_ref, i10_ref, i11_ref,
                w00_ref, w01_ref, w10_ref, w11_ref):
    q = ql_ref[...]
    offxy = jnp.dot(q, woff_ref[...], preferred_element_type=jnp.float32) + boff_ref[...]
    ox = offxy[:, :128]
    oy = offxy[:, 128:]
    al = jnp.dot(q, wat_ref[...], preferred_element_type=jnp.float32) + bat_ref[...]
    e = jnp.exp(al - jnp.max(al, axis=1, keepdims=True))
    s = jnp.dot(e, g_ref[...], preferred_element_type=jnp.float32)
    at = e / s

    x = refx_ref[...] * wlf_ref[...] + ox - 0.5
    y = refy_ref[...] * hlf_ref[...] + oy - 0.5
    x0f = jnp.floor(x)
    y0f = jnp.floor(y)
    wx = x - x0f
    wy = y - y0f
    wlm1 = wlm1_ref[...]
    hlm1 = hlm1_ref[...]
    x0 = jnp.clip(x0f.astype(jnp.int32), 0, wlm1)
    x1 = jnp.minimum(x0 + 1, wlm1)
    y0 = jnp.clip(y0f.astype(jnp.int32), 0, hlm1)
    y1 = jnp.minimum(y0 + 1, hlm1)

    wli = wlm1 + 1
    base = bb_ref[...] + start_ref[...] * 8 + hvec_ref[...]
    r0 = y0 * (wli * 8)
    r1 = y1 * (wli * 8)
    c0 = x0 * 8
    c1 = x1 * 8
    i00_ref[...] = base + r0 + c0
    i01_ref[...] = base + r0 + c1
    i10_ref[...] = base + r1 + c0
    i11_ref[...] = base + r1 + c1
    mx = 1.0 - wx
    my = 1.0 - wy
    w00_ref[...] = at * mx * my
    w01_ref[...] = at * wx * my
    w10_ref[...] = at * mx * wy
    w11_ref[...] = at * wx * wy


def _qside(ql, refx, refy, bbase, woffp, boffp, wat, bat, consts):
    bm = 1024
    g, wlf, hlf, wlm1, hlm1, start, hvec = consts
    vec_spec = pl.BlockSpec((1, 128), lambda i: (0, 0))
    outs = pl.pallas_call(
        _qside_body,
        grid=(_QROWS // bm,),
        in_specs=[pl.BlockSpec((bm, _C), lambda i: (i, 0)),
                  pl.BlockSpec((bm, 1), lambda i: (i, 0)),
                  pl.BlockSpec((bm, 1), lambda i: (i, 0)),
                  pl.BlockSpec((bm, 1), lambda i: (i, 0)),
                  pl.BlockSpec((_C, _C), lambda i: (0, 0)),
                  pl.BlockSpec((1, _C), lambda i: (0, 0)),
                  pl.BlockSpec((_C, 128), lambda i: (0, 0)),
                  vec_spec,
                  pl.BlockSpec((128, 128), lambda i: (0, 0)),
                  vec_spec, vec_spec, vec_spec, vec_spec, vec_spec, vec_spec],
        out_specs=[pl.BlockSpec((bm, 128), lambda i: (i, 0))] * 8,
        out_shape=[jax.ShapeDtypeStruct((_QROWS, 128), jnp.int32)] * 4
                + [jax.ShapeDtypeStruct((_QROWS, 128), jnp.float32)] * 4,
    )(ql, refx, refy, bbase, woffp, boffp, wat, bat, g, wlf, hlf,
      wlm1, hlm1, start, hvec)
    return outs


def _ln_in(x, g, b):
    m = jnp.mean(x, axis=1, keepdims=True)
    v = jnp.mean((x - m) ** 2, axis=1, keepdims=True)
    return (x - m) * jax.lax.rsqrt(v + 1e-5) * g + b


def _post_body(samp_ref, ql_ref, wo_ref, bo_ref, w1_ref, b1_ref, w2_ref,
               b2_ref, ln1g_ref, ln1b_ref, ln2g_ref, ln2b_ref, o_ref):
    o = jnp.dot(samp_ref[...], wo_ref[...],
                preferred_element_type=jnp.float32) + bo_ref[...]
    x = _ln_in(ql_ref[...] + o, ln1g_ref[...], ln1b_ref[...])
    h = jnp.maximum(jnp.dot(x, w1_ref[...],
                            preferred_element_type=jnp.float32) + b1_ref[...], 0.0)
    f = jnp.dot(h, w2_ref[...], preferred_element_type=jnp.float32) + b2_ref[...]
    o_ref[...] = _ln_in(x + f, ln2g_ref[...], ln2b_ref[...])


def _post(samp, ql, p, rowperm=None):
    bm = 1024
    wo = p['Wo'] if rowperm is None else p['Wo'][rowperm]
    row = lambda a: a.reshape(1, -1)
    vspec = pl.BlockSpec((1, _C), lambda i: (0, 0))
    return pl.pallas_call(
        _post_body,
        grid=(_QROWS // bm,),
        in_specs=[pl.BlockSpec((bm, _C), lambda i: (i, 0)),
                  pl.BlockSpec((bm, _C), lambda i: (i, 0)),
                  pl.BlockSpec((_C, _C), lambda i: (0, 0)),
                  vspec,
                  pl.BlockSpec((_C, 4 * _C), lambda i: (0, 0)),
                  pl.BlockSpec((1, 4 * _C), lambda i: (0, 0)),
                  pl.BlockSpec((4 * _C, _C), lambda i: (0, 0)),
                  vspec, vspec, vspec, vspec, vspec],
        out_specs=pl.BlockSpec((bm, _C), lambda i: (i, 0)),
        out_shape=jax.ShapeDtypeStruct((_QROWS, _C), jnp.float32),
    )(samp, ql, wo, row(p['bo']), p['W1'], row(p['b1']), p['W2'],
      row(p['b2']), row(p['ln1g']), row(p['ln1b']), row(p['ln2g']),
      row(p['ln2b']))


def _proj_body(qe_ref, top_ref, w1_ref, w2_ref, b_ref, uw_ref, ub_ref,
               wkv_ref, bkv_ref, proj_ref, unc_ref, khvh_ref):
    proj = (jnp.dot(qe_ref[...], w1_ref[...], preferred_element_type=jnp.float32)
            + jnp.dot(top_ref[...], w2_ref[...], preferred_element_type=jnp.float32)
            + b_ref[...])
    proj_ref[...] = proj
    unc_ref[...] = jnp.sum(proj * uw_ref[...], axis=1, keepdims=True) + ub_ref[...]
    khvh_ref[...] = jnp.dot(proj, wkv_ref[...],
                            preferred_element_type=jnp.float32) + bkv_ref[...]


def _proj(qe, top, pw, pb, uw, ub, wkv, bkv):
    bm = 1024
    return pl.pallas_call(
        _proj_body,
        grid=(_QROWS // bm,),
        in_specs=[pl.BlockSpec((bm, _C), lambda i: (i, 0)),
                  pl.BlockSpec((bm, _C), lambda i: (i, 0)),
                  pl.BlockSpec((_C, _C), lambda i: (0, 0)),
                  pl.BlockSpec((_C, _C), lambda i: (0, 0)),
                  pl.BlockSpec((1, _C), lambda i: (0, 0)),
                  pl.BlockSpec((1, _C), lambda i: (0, 0)),
                  pl.BlockSpec((1, 1), lambda i: (0, 0)),
                  pl.BlockSpec((_C, 4 * _C), lambda i: (0, 0)),
                  pl.BlockSpec((1, 4 * _C), lambda i: (0, 0))],
        out_specs=[pl.BlockSpec((bm, _C), lambda i: (i, 0)),
                   pl.BlockSpec((bm, 1), lambda i: (i, 0)),
                   pl.BlockSpec((bm, 4 * _C), lambda i: (i, 0))],
        out_shape=[jax.ShapeDtypeStruct((_QROWS, _C), jnp.float32),
                   jax.ShapeDtypeStruct((_QROWS, 1), jnp.float32),
                   jax.ShapeDtypeStruct((_QROWS, 4 * _C), jnp.float32)],
    )(qe, top, pw[:_C], pw[_C:], pb.reshape(1, _C), uw.reshape(1, _C),
      ub.reshape(1, 1), wkv, bkv.reshape(1, 4 * _C))


def _dec_body(qq_ref, kv_ref, ghk_ref, gs_ref, bk_ref, *wrefs):
    qq = qq_ref[...]
    for layer in range(2):
        (wq, bq, wo, bo, w1, b1, w2, b2, l1g, l1b, l2g, l2b) = \
            wrefs[layer * 12:(layer + 1) * 12]
        qh = jnp.dot(qq, wq[...], preferred_element_type=jnp.float32) + bq[...]
        S = jnp.zeros((qq.shape[0], 128), jnp.float32)
        for k in range(_K):
            prod = qh * kv_ref[2 * layer, :, k, :]
            S = S + jnp.dot(prod, ghk_ref[k], preferred_element_type=jnp.float32)
        S = S * (1.0 / np.sqrt(_DH).astype(np.float32))
        e = jnp.exp(S - jnp.max(S, axis=1, keepdims=True))
        den = jnp.dot(e, gs_ref[...], preferred_element_type=jnp.float32)
        at = e / den
        o = jnp.zeros((qq.shape[0], _C), jnp.float32)
        for k in range(_K):
            ab = jnp.dot(at, bk_ref[k], preferred_element_type=jnp.float32)
            o = o + ab * kv_ref[2 * layer + 1, :, k, :]
        o = jnp.dot(o, wo[...], preferred_element_type=jnp.float32) + bo[...]
        x = _ln_in(qq + o, l1g[...], l1b[...])
        h = jnp.maximum(jnp.dot(x, w1[...],
                                preferred_element_type=jnp.float32) + b1[...], 0.0)
        f = jnp.dot(h, w2[...], preferred_element_type=jnp.float32) + b2[...]
        qq = _ln_in(x + f, l2g[...], l2b[...])
    fw1, fw2, fb = wrefs[24], wrefs[25], wrefs[26]
    out_ref = wrefs[27]
    out_ref[...] = (jnp.dot(qq_ref[...], fw1[...], preferred_element_type=jnp.float32)
                    + jnp.dot(qq, fw2[...], preferred_element_type=jnp.float32)
                    + fb[...])


def _decoder(qq0, kv4, ghk, gs, bks, dparams, fw, fb):
    bm = 128
    rows = _B * _N
    row = lambda a: a.reshape(1, -1)
    vspec = pl.BlockSpec((1, _C), lambda i: (0, 0))
    in_specs = [pl.BlockSpec((bm, _C), lambda i: (i, 0)),
                pl.BlockSpec((4, bm, _K, _C), lambda i: (0, i, 0, 0)),
                pl.BlockSpec((_K, _C, 128), lambda i: (0, 0, 0)),
                pl.BlockSpec((128, 128), lambda i: (0, 0)),
                pl.BlockSpec((_K, 128, _C), lambda i: (0, 0, 0))]
    args = [qq0, kv4, ghk, gs, bks]
    for p in dparams:
        in_specs += [pl.BlockSpec((_C, _C), lambda i: (0, 0)), vspec,
                     pl.BlockSpec((_C, _C), lambda i: (0, 0)), vspec,
                     pl.BlockSpec((_C, 4 * _C), lambda i: (0, 0)),
                     pl.BlockSpec((1, 4 * _C), lambda i: (0, 0)),
                     pl.BlockSpec((4 * _C, _C), lambda i: (0, 0)),
                     vspec, vspec, vspec, vspec, vspec]
        args += [p['Wq'], row(p['bq']), p['Wo'], row(p['bo']), p['W1'],
                 row(p['b1']), p['W2'], row(p['b2']), row(p['ln1g']),
                 row(p['ln1b']), row(p['ln2g']), row(p['ln2b'])]
    in_specs += [pl.BlockSpec((_C, _C), lambda i: (0, 0)),
                 pl.BlockSpec((_C, _C), lambda i: (0, 0)), vspec]
    args += [fw[:_C], fw[_C:], row(fb)]
    return pl.pallas_call(
        _dec_body,
        grid=(rows // bm,),
        in_specs=in_specs,
        out_specs=pl.BlockSpec((bm, _C), lambda i: (i, 0)),
        out_shape=jax.ShapeDtypeStruct((rows, _C), jnp.float32),
    )(*args)


_NW = 32
_CB = 8
_GPW = 1024
_RPW = 2048


_NCH = _GPW // _CB


def _bag_body(table, idxh, wh, out, idx_v, w_v, grows, out_v, sem_g, sem_s,
              sem_o):
    wid = lax.axis_index("s") * 2 + lax.axis_index("c")

    def stage_idx(cc):
        @pl.when(cc < _NCH)
        def _():
            s4 = lax.rem(cc, 4)
            pltpu.async_copy(idxh.at[wid, pl.ds(cc * _CB, _CB), :],
                             idx_v.at[s4], sem_s)
            pltpu.async_copy(wh.at[wid, pl.ds(cc * _CB, _CB), :],
                             w_v.at[s4], sem_s)

    def wait_stage(cc):
        @pl.when(cc < _NCH)
        def _():
            s4 = lax.rem(cc, 4)
            pltpu.make_async_copy(idxh.at[wid, pl.ds(cc * _CB, _CB), :],
                                  idx_v.at[s4], sem_s).wait()
            pltpu.make_async_copy(wh.at[wid, pl.ds(cc * _CB, _CB), :],
                                  w_v.at[s4], sem_s).wait()

    def fire_gathers(cc):
        @pl.when(cc < _NCH)
        def _():
            s4 = lax.rem(cc, 4)
            s2 = lax.rem(cc, 2)
            for g in range(_CB):
                pltpu.async_copy(table.at[idx_v.at[s4, g]],
                                 grows.at[s2, g], sem_g)

    stage_idx(0)
    stage_idx(1)
    wait_stage(0)
    fire_gathers(0)

    def outer(c, carry):
        s4 = lax.rem(c, 4)
        s2 = lax.rem(c, 2)
        stage_idx(c + 2)
        wait_stage(c + 1)
        fire_gathers(c + 1)
        for g in range(_CB):
            pltpu.make_async_copy(table.at[idx_v.at[s4, g]],
                                  grows.at[s2, g], sem_g).wait()

        @pl.when(c >= 2)
        def _():
            pltpu.make_async_copy(
                out_v.at[s2],
                out.at[pl.ds(wid * _RPW + (c - 2) * 2 * _CB, 2 * _CB), :],
                sem_o).wait()
        for g in range(_CB):
            for half in range(2):
                off = half * 64
                wvs = [w_v[s4, g, pl.ds(off + k * 16, 16)] for k in range(4)]
                z = jnp.zeros((16,), jnp.float32)
                a0 = [z, z, z, z]
                a1 = [z, z, z, z]
                for j in range(64):
                    k = j // 16
                    s = wvs[k][j % 16]
                    a0[k] = a0[k] + s * grows[s2, g, off + j, pl.ds(0, 16)]
                    a1[k] = a1[k] + s * grows[s2, g, off + j, pl.ds(16, 16)]
                out_v[s2, 2 * g + half, pl.ds(0, 16)] = (a0[0] + a0[1]) + (a0[2] + a0[3])
                out_v[s2, 2 * g + half, pl.ds(16, 16)] = (a1[0] + a1[1]) + (a1[2] + a1[3])
        pltpu.async_copy(out_v.at[s2],
                         out.at[pl.ds(wid * _RPW + c * 2 * _CB, 2 * _CB), :],
                         sem_o)
        return carry

    lax.fori_loop(0, _NCH, outer, None)
    for cc in (_NCH - 2, _NCH - 1):
        pltpu.make_async_copy(
            out_v.at[cc % 2],
            out.at[pl.ds(wid * _RPW + cc * 2 * _CB, 2 * _CB), :],
            sem_o).wait()


@functools.partial(
    pl.kernel,
    mesh=plsc.VectorSubcoreMesh(core_axis_name="c", subcore_axis_name="s"),
    compiler_params=pltpu.CompilerParams(use_tc_tiling_on_sc=False),
    out_type=jax.ShapeDtypeStruct((_OROWS, _DH), jnp.float32),
    scratch_types=[
        pltpu.VMEM((4, _CB, 128), jnp.int32),
        pltpu.VMEM((4, _CB, 128), jnp.float32),
        pltpu.VMEM((2, _CB, 128, _DH), jnp.float32),
        pltpu.VMEM((2, 2 * _CB, _DH), jnp.float32),
        pltpu.SemaphoreType.DMA,
        pltpu.SemaphoreType.DMA,
        pltpu.SemaphoreType.DMA,
    ],
)
def _bag(table, idxh, wh, out, idx_v, w_v, grows, out_v, sem_g, sem_s, sem_o):
    _bag_body(table, idxh, wh, out, idx_v, w_v, grows, out_v, sem_g, sem_s,
              sem_o)


_UNPACK_ROWPERM = np.concatenate(
    [h * 32 + np.concatenate([np.arange(16) * 2, np.arange(16) * 2 + 1])
     for h in range(_NHEAD)])


def _lane_consts():
    lanes = np.arange(128)
    h = lanes // 16
    l = (lanes % 16) // 4
    wl = np.array([s[1] for s in _LEVEL_SHAPES], np.float32)[l]
    hl = np.array([s[0] for s in _LEVEL_SHAPES], np.float32)[l]
    start = np.array(_START, np.int32)[l]
    g = (lanes[:, None] // 16 == lanes[None, :] // 16).astype(np.float32)
    return (jnp.asarray(g),
            jnp.asarray(wl.reshape(1, 128)),
            jnp.asarray(hl.reshape(1, 128)),
            jnp.asarray((wl - 1).astype(np.int32).reshape(1, 128)),
            jnp.asarray((hl - 1).astype(np.int32).reshape(1, 128)),
            jnp.asarray(start.reshape(1, 128)),
            jnp.asarray(h.astype(np.int32).reshape(1, 128)))


def _dec_consts():
    i = np.arange(_C)
    ghk = np.zeros((_K, _C, 128), np.float32)
    for k in range(_K):
        ghk[k, i, (i // _DH) * _K + k] = 1.0
    lanes = np.arange(128)
    gs = (lanes[:, None] // _K == lanes[None, :] // _K).astype(np.float32)
    bk = np.zeros((_K, 128, _C), np.float32)
    for k in range(_K):
        for h in range(_NHEAD):
            bk[k, h * _K + k, h * _DH:(h + 1) * _DH] = 1.0
    return jnp.asarray(ghk), jnp.asarray(gs), jnp.asarray(bk)


def kernel(q_t, h_t, c_t, params):
    idx = _topk(c_t)
    xs = (idx % _WP).astype(jnp.float32) * _STRIDE + _STRIDE / 2.0
    ys = (idx // _WP).astype(jnp.float32) * _STRIDE + _STRIDE / 2.0
    tloc = jnp.stack([xs, ys], -1)
    norm = jnp.clip(tloc / 1024.0, 0.0, 1.0)

    img = h_t.reshape(_B, _HP, _WP, _C)
    f1 = h_t
    f2 = img.reshape(_B, 32, 2, 32, 2, _C).mean((2, 4)).reshape(_B, 1024, _C)
    f3 = img.reshape(_B, 16, 4, 16, 4, _C).mean((2, 4)).reshape(_B, 256, _C)
    f4 = f1.reshape(_B, 512, 8, 32, 8).mean((2, 4)).reshape(_B, _C, 64).transpose(0, 2, 1)
    fs = jnp.concatenate([f1, f2, f3, f4], 1).reshape(_B * _LVL, _C)

    enc = params['enc']
    wv = jnp.concatenate([enc[0]['Wv'], enc[1]['Wv']], axis=1)
    bv = jnp.concatenate([enc[0]['bv'], enc[1]['bv']], axis=0)
    vall = _mm(fs, wv, bv, bm=1360)

    consts = _lane_consts()
    refx = norm[..., 0].reshape(_QROWS, 1)
    refy = norm[..., 1].reshape(_QROWS, 1)
    bbase = jnp.repeat(jnp.arange(_B, dtype=jnp.int32) * (_LVL * _NHEAD),
                       _N * _K).reshape(_QROWS, 1)
    qe = jnp.broadcast_to(q_t[:, :, None, :], (_B, _N, _K, _C)).reshape(_QROWS, _C)

    perm = np.array([h * 32 + l * 8 + p * 2 + xy
                     for xy in range(2) for h in range(_NHEAD)
                     for l in range(_NUM_LEVEL) for p in range(_NUM_POINTS)])
    ql = qe
    for li, p in enumerate(enc):
        woffp = p['Woff'][:, perm]
        boffp = p['boff'][perm].reshape(1, 256)
        outs = _qside(ql, refx, refy, bbase, woffp, boffp, p['Wat'],
                      p['bat'].reshape(1, 128), consts)
        i00, i01, i10, i11, w00, w01, w10, w11 = outs
        idxs = jnp.concatenate(
            [a.reshape(_OROWS, 16) for a in (i00, i01, i10, i11)], axis=1)
        ws = jnp.concatenate(
            [a.reshape(_OROWS, 16) for a in (w00, w01, w10, w11)], axis=1)
        table = vall[:, li * _C:(li + 1) * _C].reshape(_TROWS, _DH)
        samp = _bag(table, idxs.reshape(_NW, _GPW, 128),
                    ws.reshape(_NW, _GPW, 128))
        ql = _post(samp.reshape(_QROWS, _C), ql, p)

    dec = params['dec']
    wkv = jnp.concatenate([dec[0]['Wk'], dec[0]['Wv'],
                           dec[1]['Wk'], dec[1]['Wv']], axis=1)
    bkv = jnp.concatenate([dec[0]['bk'], dec[0]['bv'],
                           dec[1]['bk'], dec[1]['bv']], axis=0)
    proj, unc, khvh = _proj(qe, ql, params['proj_W'], params['proj_b'],
                            params['unc_W'], params['unc_b'], wkv, bkv)

    kv4 = khvh.reshape(_B * _N, _K, 4, _C).transpose(2, 0, 1, 3)
    ghk, gs, bks = _dec_consts()
    out = _decoder(q_t.reshape(_B * _N, _C), kv4, ghk, gs, bks, dec,
                   params['fusion_W'], params['fusion_b'])

    return (out.reshape(_B, _N, _C), unc.reshape(_B, _N, _K), tloc)


# speedup vs baseline: 1.0269x; 1.0149x over previous
import functools

import numpy as np

import jax
import jax.numpy as jnp
from jax import lax
from jax.experimental import pallas as pl
from jax.experimental.pallas import tpu as pltpu
from jax.experimental.pallas import tpu_sc as plsc

_B = 4
_N = 128
_C = 256
_NHEAD = 8
_DH = _C // _NHEAD
_STRIDE = 16
_HP = 64
_WP = 64
_P = _HP * _WP
_K = 16
_NUM_LEVEL = 4
_NUM_POINTS = 4
_LEVEL_SHAPES = [(64, 64), (32, 32), (16, 16), (8, 8)]
_START = [0, 4096, 5120, 5376]
_LVL = 5440
_QROWS = _B * _N * _K
_OROWS = _QROWS * _NHEAD
_TERMS = 64
_TROWS = _B * _LVL * _NHEAD


def _mm_body(x_ref, w_ref, b_ref, o_ref):
    o_ref[...] = jnp.dot(x_ref[...], w_ref[...],
                         preferred_element_type=jnp.float32) + b_ref[...]


def _mm(x, w, b, bm):
    R, Kd = x.shape
    F = w.shape[1]
    return pl.pallas_call(
        _mm_body,
        grid=(R // bm,),
        in_specs=[pl.BlockSpec((bm, Kd), lambda i: (i, 0)),
                  pl.BlockSpec((Kd, F), lambda i: (0, 0)),
                  pl.BlockSpec((1, F), lambda i: (0, 0))],
        out_specs=pl.BlockSpec((bm, F), lambda i: (i, 0)),
        out_shape=jax.ShapeDtypeStruct((R, F), jnp.float32),
    )(x, w, b.reshape(1, F))


def _topk_body(c_ref, idx_ref):
    x = c_ref[0]
    iota = lax.broadcasted_iota(jnp.int32, x.shape, 1)
    cols = []
    for _ in range(_K):
        m = jnp.max(x, axis=1, keepdims=True)
        am = jnp.min(jnp.where(x == m, iota, x.shape[1]), axis=1, keepdims=True)
        cols.append(am)
        x = jnp.where(iota == am, -jnp.inf, x)
    idx_ref[0] = jnp.concatenate(cols, axis=1)


def _topk(c_t):
    return pl.pallas_call(
        _topk_body,
        grid=(_B,),
        in_specs=[pl.BlockSpec((1, _N, _P), lambda i: (i, 0, 0))],
        out_specs=pl.BlockSpec((1, _N, _K), lambda i: (i, 0, 0)),
        out_shape=jax.ShapeDtypeStruct((_B, _N, _K), jnp.int32),
    )(c_t)


def _qside_body(ql_ref, refx_ref, refy_ref, bb_ref, woff_ref, boff_ref,
                wat_ref, bat_ref, g_ref, wlf_ref, hlf_ref, wlm1_ref, hlm1_ref,
                start_ref, hvec_ref,
                i00_ref, i01_ref, i10_ref, i11_ref,
                w00_ref, w01_ref, w10_ref, w11_ref):
    q = ql_ref[...]
    offxy = jnp.dot(q, woff_ref[...], preferred_element_type=jnp.float32) + boff_ref[...]
    ox = offxy[:, :128]
    oy = offxy[:, 128:]
    al = jnp.dot(q, wat_ref[...], preferred_element_type=jnp.float32) + bat_ref[...]
    e = jnp.exp(al - jnp.max(al, axis=1, keepdims=True))
    s = jnp.dot(e, g_ref[...], preferred_element_type=jnp.float32)
    at = e / s

    x = refx_ref[...] * wlf_ref[...] + ox - 0.5
    y = refy_ref[...] * hlf_ref[...] + oy - 0.5
    x0f = jnp.floor(x)
    y0f = jnp.floor(y)
    wx = x - x0f
    wy = y - y0f
    wlm1 = wlm1_ref[...]
    hlm1 = hlm1_ref[...]
    x0 = jnp.clip(x0f.astype(jnp.int32), 0, wlm1)
    x1 = jnp.minimum(x0 + 1, wlm1)
    y0 = jnp.clip(y0f.astype(jnp.int32), 0, hlm1)
    y1 = jnp.minimum(y0 + 1, hlm1)

    wli = wlm1 + 1
    base = bb_ref[...] + start_ref[...] * 8 + hvec_ref[...]
    r0 = y0 * (wli * 8)
    r1 = y1 * (wli * 8)
    c0 = x0 * 8
    c1 = x1 * 8
    i00_ref[...] = base + r0 + c0
    i01_ref[...] = base + r0 + c1
    i10_ref[...] = base + r1 + c0
    i11_ref[...] = base + r1 + c1
    mx = 1.0 - wx
    my = 1.0 - wy
    w00_ref[...] = at * mx * my
    w01_ref[...] = at * wx * my
    w10_ref[...] = at * mx * wy
    w11_ref[...] = at * wx * wy


def _qside(ql, refx, refy, bbase, woffp, boffp, wat, bat, consts):
    bm = 1024
    g, wlf, hlf, wlm1, hlm1, start, hvec = consts
    vec_spec = pl.BlockSpec((1, 128), lambda i: (0, 0))
    outs = pl.pallas_call(
        _qside_body,
        grid=(_QROWS // bm,),
        in_specs=[pl.BlockSpec((bm, _C), lambda i: (i, 0)),
                  pl.BlockSpec((bm, 1), lambda i: (i, 0)),
                  pl.BlockSpec((bm, 1), lambda i: (i, 0)),
                  pl.BlockSpec((bm, 1), lambda i: (i, 0)),
                  pl.BlockSpec((_C, _C), lambda i: (0, 0)),
                  pl.BlockSpec((1, _C), lambda i: (0, 0)),
                  pl.BlockSpec((_C, 128), lambda i: (0, 0)),
                  vec_spec,
                  pl.BlockSpec((128, 128), lambda i: (0, 0)),
                  vec_spec, vec_spec, vec_spec, vec_spec, vec_spec, vec_spec],
        out_specs=[pl.BlockSpec((bm, 128), lambda i: (i, 0))] * 8,
        out_shape=[jax.ShapeDtypeStruct((_QROWS, 128), jnp.int32)] * 4
                + [jax.ShapeDtypeStruct((_QROWS, 128), jnp.float32)] * 4,
    )(ql, refx, refy, bbase, woffp, boffp, wat, bat, g, wlf, hlf,
      wlm1, hlm1, start, hvec)
    return outs


def _ln_in(x, g, b):
    m = jnp.mean(x, axis=1, keepdims=True)
    v = jnp.mean((x - m) ** 2, axis=1, keepdims=True)
    return (x - m) * jax.lax.rsqrt(v + 1e-5) * g + b


def _post_body(samp_ref, ql_ref, wo_ref, bo_ref, w1_ref, b1_ref, w2_ref,
               b2_ref, ln1g_ref, ln1b_ref, ln2g_ref, ln2b_ref, o_ref):
    o = jnp.dot(samp_ref[...], wo_ref[...],
                preferred_element_type=jnp.float32) + bo_ref[...]
    x = _ln_in(ql_ref[...] + o, ln1g_ref[...], ln1b_ref[...])
    h = jnp.maximum(jnp.dot(x, w1_ref[...],
                            preferred_element_type=jnp.float32) + b1_ref[...], 0.0)
    f = jnp.dot(h, w2_ref[...], preferred_element_type=jnp.float32) + b2_ref[...]
    o_ref[...] = _ln_in(x + f, ln2g_ref[...], ln2b_ref[...])


def _post(samp, ql, p, rowperm=None):
    bm = 1024
    wo = p['Wo'] if rowperm is None else p['Wo'][rowperm]
    row = lambda a: a.reshape(1, -1)
    vspec = pl.BlockSpec((1, _C), lambda i: (0, 0))
    return pl.pallas_call(
        _post_body,
        grid=(_QROWS // bm,),
        in_specs=[pl.BlockSpec((bm, _C), lambda i: (i, 0)),
                  pl.BlockSpec((bm, _C), lambda i: (i, 0)),
                  pl.BlockSpec((_C, _C), lambda i: (0, 0)),
                  vspec,
                  pl.BlockSpec((_C, 4 * _C), lambda i: (0, 0)),
                  pl.BlockSpec((1, 4 * _C), lambda i: (0, 0)),
                  pl.BlockSpec((4 * _C, _C), lambda i: (0, 0)),
                  vspec, vspec, vspec, vspec, vspec],
        out_specs=pl.BlockSpec((bm, _C), lambda i: (i, 0)),
        out_shape=jax.ShapeDtypeStruct((_QROWS, _C), jnp.float32),
    )(samp, ql, wo, row(p['bo']), p['W1'], row(p['b1']), p['W2'],
      row(p['b2']), row(p['ln1g']), row(p['ln1b']), row(p['ln2g']),
      row(p['ln2b']))


def _proj_body(qe_ref, top_ref, w1_ref, w2_ref, b_ref, uw_ref, ub_ref,
               wkv_ref, bkv_ref, proj_ref, unc_ref, khvh_ref):
    proj = (jnp.dot(qe_ref[...], w1_ref[...], preferred_element_type=jnp.float32)
            + jnp.dot(top_ref[...], w2_ref[...], preferred_element_type=jnp.float32)
            + b_ref[...])
    proj_ref[...] = proj
    unc_ref[...] = jnp.sum(proj * uw_ref[...], axis=1, keepdims=True) + ub_ref[...]
    khvh_ref[...] = jnp.dot(proj, wkv_ref[...],
                            preferred_element_type=jnp.float32) + bkv_ref[...]


def _proj(qe, top, pw, pb, uw, ub, wkv, bkv):
    bm = 1024
    return pl.pallas_call(
        _proj_body,
        grid=(_QROWS // bm,),
        in_specs=[pl.BlockSpec((bm, _C), lambda i: (i, 0)),
                  pl.BlockSpec((bm, _C), lambda i: (i, 0)),
                  pl.BlockSpec((_C, _C), lambda i: (0, 0)),
                  pl.BlockSpec((_C, _C), lambda i: (0, 0)),
                  pl.BlockSpec((1, _C), lambda i: (0, 0)),
                  pl.BlockSpec((1, _C), lambda i: (0, 0)),
                  pl.BlockSpec((1, 1), lambda i: (0, 0)),
                  pl.BlockSpec((_C, 4 * _C), lambda i: (0, 0)),
                  pl.BlockSpec((1, 4 * _C), lambda i: (0, 0))],
        out_specs=[pl.BlockSpec((bm, _C), lambda i: (i, 0)),
                   pl.BlockSpec((bm, 1), lambda i: (i, 0)),
                   pl.BlockSpec((bm, 4 * _C), lambda i: (i, 0))],
        out_shape=[jax.ShapeDtypeStruct((_QROWS, _C), jnp.float32),
                   jax.ShapeDtypeStruct((_QROWS, 1), jnp.float32),
                   jax.ShapeDtypeStruct((_QROWS, 4 * _C), jnp.float32)],
    )(qe, top, pw[:_C], pw[_C:], pb.reshape(1, _C), uw.reshape(1, _C),
      ub.reshape(1, 1), wkv, bkv.reshape(1, 4 * _C))


def _dec_body(qq_ref, kv_ref, ghk_ref, gs_ref, bk_ref, *wrefs):
    qq = qq_ref[...]
    for layer in range(2):
        (wq, bq, wo, bo, w1, b1, w2, b2, l1g, l1b, l2g, l2b) = \
            wrefs[layer * 12:(layer + 1) * 12]
        qh = jnp.dot(qq, wq[...], preferred_element_type=jnp.float32) + bq[...]
        S = jnp.zeros((qq.shape[0], 128), jnp.float32)
        for k in range(_K):
            prod = qh * kv_ref[2 * layer, :, k, :]
            S = S + jnp.dot(prod, ghk_ref[k], preferred_element_type=jnp.float32)
        S = S * (1.0 / np.sqrt(_DH).astype(np.float32))
        e = jnp.exp(S - jnp.max(S, axis=1, keepdims=True))
        den = jnp.dot(e, gs_ref[...], preferred_element_type=jnp.float32)
        at = e / den
        o = jnp.zeros((qq.shape[0], _C), jnp.float32)
        for k in range(_K):
            ab = jnp.dot(at, bk_ref[k], preferred_element_type=jnp.float32)
            o = o + ab * kv_ref[2 * layer + 1, :, k, :]
        o = jnp.dot(o, wo[...], preferred_element_type=jnp.float32) + bo[...]
        x = _ln_in(qq + o, l1g[...], l1b[...])
        h = jnp.maximum(jnp.dot(x, w1[...],
                                preferred_element_type=jnp.float32) + b1[...], 0.0)
        f = jnp.dot(h, w2[...], preferred_element_type=jnp.float32) + b2[...]
        qq = _ln_in(x + f, l2g[...], l2b[...])
    fw1, fw2, fb = wrefs[24], wrefs[25], wrefs[26]
    out_ref = wrefs[27]
    out_ref[...] = (jnp.dot(qq_ref[...], fw1[...], preferred_element_type=jnp.float32)
                    + jnp.dot(qq, fw2[...], preferred_element_type=jnp.float32)
                    + fb[...])


def _decoder(qq0, kv4, ghk, gs, bks, dparams, fw, fb):
    bm = 128
    rows = _B * _N
    row = lambda a: a.reshape(1, -1)
    vspec = pl.BlockSpec((1, _C), lambda i: (0, 0))
    in_specs = [pl.BlockSpec((bm, _C), lambda i: (i, 0)),
                pl.BlockSpec((4, bm, _K, _C), lambda i: (0, i, 0, 0)),
                pl.BlockSpec((_K, _C, 128), lambda i: (0, 0, 0)),
                pl.BlockSpec((128, 128), lambda i: (0, 0)),
                pl.BlockSpec((_K, 128, _C), lambda i: (0, 0, 0))]
    args = [qq0, kv4, ghk, gs, bks]
    for p in dparams:
        in_specs += [pl.BlockSpec((_C, _C), lambda i: (0, 0)), vspec,
                     pl.BlockSpec((_C, _C), lambda i: (0, 0)), vspec,
                     pl.BlockSpec((_C, 4 * _C), lambda i: (0, 0)),
                     pl.BlockSpec((1, 4 * _C), lambda i: (0, 0)),
                     pl.BlockSpec((4 * _C, _C), lambda i: (0, 0)),
                     vspec, vspec, vspec, vspec, vspec]
        args += [p['Wq'], row(p['bq']), p['Wo'], row(p['bo']), p['W1'],
                 row(p['b1']), p['W2'], row(p['b2']), row(p['ln1g']),
                 row(p['ln1b']), row(p['ln2g']), row(p['ln2b'])]
    in_specs += [pl.BlockSpec((_C, _C), lambda i: (0, 0)),
                 pl.BlockSpec((_C, _C), lambda i: (0, 0)), vspec]
    args += [fw[:_C], fw[_C:], row(fb)]
    return pl.pallas_call(
        _dec_body,
        grid=(rows // bm,),
        in_specs=in_specs,
        out_specs=pl.BlockSpec((bm, _C), lambda i: (i, 0)),
        out_shape=jax.ShapeDtypeStruct((rows, _C), jnp.float32),
    )(*args)


_NW = 32
_CB = 8
_GPW = 1024
_RPW = 2048


_NCH = _GPW // _CB


def _bag_body(table, idxh, wh, out, idx_v, w_v, grows, out_v, sem_g, sem_s,
              sem_o):
    wid = lax.axis_index("s") * 2 + lax.axis_index("c")

    def stage_idx(cc):
        @pl.when(cc < _NCH)
        def _():
            s4 = lax.rem(cc, 4)
            pltpu.async_copy(idxh.at[wid, pl.ds(cc * _CB, _CB), :],
                             idx_v.at[s4], sem_s)
            pltpu.async_copy(wh.at[wid, pl.ds(cc * _CB, _CB), :],
                             w_v.at[s4], sem_s)

    def wait_stage(cc):
        @pl.when(cc < _NCH)
        def _():
            s4 = lax.rem(cc, 4)
            pltpu.make_async_copy(idxh.at[wid, pl.ds(cc * _CB, _CB), :],
                                  idx_v.at[s4], sem_s).wait()
            pltpu.make_async_copy(wh.at[wid, pl.ds(cc * _CB, _CB), :],
                                  w_v.at[s4], sem_s).wait()

    def fire_gathers(cc):
        @pl.when(cc < _NCH)
        def _():
            s4 = lax.rem(cc, 4)
            s3 = lax.rem(cc, 3)
            for g in range(_CB):
                pltpu.async_copy(table.at[idx_v.at[s4, g]],
                                 grows.at[s3, g], sem_g)

    stage_idx(0)
    stage_idx(1)
    stage_idx(2)
    wait_stage(0)
    fire_gathers(0)
    wait_stage(1)
    fire_gathers(1)

    def outer(c, carry):
        s4 = lax.rem(c, 4)
        s2 = lax.rem(c, 2)
        s3 = lax.rem(c, 3)
        stage_idx(c + 3)
        wait_stage(c + 2)
        fire_gathers(c + 2)
        for g in range(_CB):
            pltpu.make_async_copy(table.at[idx_v.at[s4, g]],
                                  grows.at[s3, g], sem_g).wait()

        @pl.when(c >= 2)
        def _():
            pltpu.make_async_copy(
                out_v.at[s2],
                out.at[pl.ds(wid * _RPW + (c - 2) * 2 * _CB, 2 * _CB), :],
                sem_o).wait()
        for g in range(_CB):
            for half in range(2):
                off = half * 64
                wvs = [w_v[s4, g, pl.ds(off + k * 16, 16)] for k in range(4)]
                z = jnp.zeros((16,), jnp.float32)
                a0 = [z, z, z, z]
                a1 = [z, z, z, z]
                for j in range(64):
                    k = j // 16
                    s = wvs[k][j % 16]
                    a0[k] = a0[k] + s * grows[s3, g, off + j, pl.ds(0, 16)]
                    a1[k] = a1[k] + s * grows[s3, g, off + j, pl.ds(16, 16)]
                out_v[s2, 2 * g + half, pl.ds(0, 16)] = (a0[0] + a0[1]) + (a0[2] + a0[3])
                out_v[s2, 2 * g + half, pl.ds(16, 16)] = (a1[0] + a1[1]) + (a1[2] + a1[3])
        pltpu.async_copy(out_v.at[s2],
                         out.at[pl.ds(wid * _RPW + c * 2 * _CB, 2 * _CB), :],
                         sem_o)
        return carry

    lax.fori_loop(0, _NCH, outer, None)
    for cc in (_NCH - 2, _NCH - 1):
        pltpu.make_async_copy(
            out_v.at[cc % 2],
            out.at[pl.ds(wid * _RPW + cc * 2 * _CB, 2 * _CB), :],
            sem_o).wait()


@functools.partial(
    pl.kernel,
    mesh=plsc.VectorSubcoreMesh(core_axis_name="c", subcore_axis_name="s"),
    compiler_params=pltpu.CompilerParams(use_tc_tiling_on_sc=False),
    out_type=jax.ShapeDtypeStruct((_OROWS, _DH), jnp.float32),
    scratch_types=[
        pltpu.VMEM((4, _CB, 128), jnp.int32),
        pltpu.VMEM((4, _CB, 128), jnp.float32),
        pltpu.VMEM((3, _CB, 128, _DH), jnp.float32),
        pltpu.VMEM((2, 2 * _CB, _DH), jnp.float32),
        pltpu.SemaphoreType.DMA,
        pltpu.SemaphoreType.DMA,
        pltpu.SemaphoreType.DMA,
    ],
)
def _bag(table, idxh, wh, out, idx_v, w_v, grows, out_v, sem_g, sem_s, sem_o):
    _bag_body(table, idxh, wh, out, idx_v, w_v, grows, out_v, sem_g, sem_s,
              sem_o)


_UNPACK_ROWPERM = np.concatenate(
    [h * 32 + np.concatenate([np.arange(16) * 2, np.arange(16) * 2 + 1])
     for h in range(_NHEAD)])


def _lane_consts():
    lanes = np.arange(128)
    h = lanes // 16
    l = (lanes % 16) // 4
    wl = np.array([s[1] for s in _LEVEL_SHAPES], np.float32)[l]
    hl = np.array([s[0] for s in _LEVEL_SHAPES], np.float32)[l]
    start = np.array(_START, np.int32)[l]
    g = (lanes[:, None] // 16 == lanes[None, :] // 16).astype(np.float32)
    return (jnp.asarray(g),
            jnp.asarray(wl.reshape(1, 128)),
            jnp.asarray(hl.reshape(1, 128)),
            jnp.asarray((wl - 1).astype(np.int32).reshape(1, 128)),
            jnp.asarray((hl - 1).astype(np.int32).reshape(1, 128)),
            jnp.asarray(start.reshape(1, 128)),
            jnp.asarray(h.astype(np.int32).reshape(1, 128)))


def _dec_consts():
    i = np.arange(_C)
    ghk = np.zeros((_K, _C, 128), np.float32)
    for k in range(_K):
        ghk[k, i, (i // _DH) * _K + k] = 1.0
    lanes = np.arange(128)
    gs = (lanes[:, None] // _K == lanes[None, :] // _K).astype(np.float32)
    bk = np.zeros((_K, 128, _C), np.float32)
    for k in range(_K):
        for h in range(_NHEAD):
            bk[k, h * _K + k, h * _DH:(h + 1) * _DH] = 1.0
    return jnp.asarray(ghk), jnp.asarray(gs), jnp.asarray(bk)


def kernel(q_t, h_t, c_t, params):
    idx = _topk(c_t)
    xs = (idx % _WP).astype(jnp.float32) * _STRIDE + _STRIDE / 2.0
    ys = (idx // _WP).astype(jnp.float32) * _STRIDE + _STRIDE / 2.0
    tloc = jnp.stack([xs, ys], -1)
    norm = jnp.clip(tloc / 1024.0, 0.0, 1.0)

    img = h_t.reshape(_B, _HP, _WP, _C)
    f1 = h_t
    f2 = img.reshape(_B, 32, 2, 32, 2, _C).mean((2, 4)).reshape(_B, 1024, _C)
    f3 = img.reshape(_B, 16, 4, 16, 4, _C).mean((2, 4)).reshape(_B, 256, _C)
    f4 = f1.reshape(_B, 512, 8, 32, 8).mean((2, 4)).reshape(_B, _C, 64).transpose(0, 2, 1)
    fs = jnp.concatenate([f1, f2, f3, f4], 1).reshape(_B * _LVL, _C)

    enc = params['enc']
    wv = jnp.concatenate([enc[0]['Wv'], enc[1]['Wv']], axis=1)
    bv = jnp.concatenate([enc[0]['bv'], enc[1]['bv']], axis=0)
    vall = _mm(fs, wv, bv, bm=1360)

    consts = _lane_consts()
    refx = norm[..., 0].reshape(_QROWS, 1)
    refy = norm[..., 1].reshape(_QROWS, 1)
    bbase = jnp.repeat(jnp.arange(_B, dtype=jnp.int32) * (_LVL * _NHEAD),
                       _N * _K).reshape(_QROWS, 1)
    qe = jnp.broadcast_to(q_t[:, :, None, :], (_B, _N, _K, _C)).reshape(_QROWS, _C)

    perm = np.array([h * 32 + l * 8 + p * 2 + xy
                     for xy in range(2) for h in range(_NHEAD)
                     for l in range(_NUM_LEVEL) for p in range(_NUM_POINTS)])
    ql = qe
    for li, p in enumerate(enc):
        woffp = p['Woff'][:, perm]
        boffp = p['boff'][perm].reshape(1, 256)
        outs = _qside(ql, refx, refy, bbase, woffp, boffp, p['Wat'],
                      p['bat'].reshape(1, 128), consts)
        i00, i01, i10, i11, w00, w01, w10, w11 = outs
        idxs = jnp.concatenate(
            [a.reshape(_OROWS, 16) for a in (i00, i01, i10, i11)], axis=1)
        ws = jnp.concatenate(
            [a.reshape(_OROWS, 16) for a in (w00, w01, w10, w11)], axis=1)
        table = vall[:, li * _C:(li + 1) * _C].reshape(_TROWS, _DH)
        samp = _bag(table, idxs.reshape(_NW, _GPW, 128),
                    ws.reshape(_NW, _GPW, 128))
        ql = _post(samp.reshape(_QROWS, _C), ql, p)

    dec = params['dec']
    wkv = jnp.concatenate([dec[0]['Wk'], dec[0]['Wv'],
                           dec[1]['Wk'], dec[1]['Wv']], axis=1)
    bkv = jnp.concatenate([dec[0]['bk'], dec[0]['bv'],
                           dec[1]['bk'], dec[1]['bv']], axis=0)
    proj, unc, khvh = _proj(qe, ql, params['proj_W'], params['proj_b'],
                            params['unc_W'], params['unc_b'], wkv, bkv)

    kv4 = khvh.reshape(_B * _N, _K, 4, _C).transpose(2, 0, 1, 3)
    ghk, gs, bks = _dec_consts()
    out = _decoder(q_t.reshape(_B * _N, _C), kv4, ghk, gs, bks, dec,
                   params['fusion_W'], params['fusion_b'])

    return (out.reshape(_B, _N, _C), unc.reshape(_B, _N, _K), tloc)
